# Initial kernel scaffold; baseline (speedup 1.0000x reference)
import jax
import jax.numpy as jnp
from jax.experimental import pallas as pl


def kernel(x, z, h, w):
    raise NotImplementedError("write your pallas kernel here")


# speedup vs baseline: 1.3813x; 1.3813x over previous
import functools

import jax
import jax.numpy as jnp
from jax import lax
from jax.experimental import pallas as pl
from jax.experimental.pallas import tpu as pltpu
from jax.experimental.pallas import tpu_sc as plsc

LANES = 16
NC, NS = 2, 16
NW = NC * NS


@functools.lru_cache(maxsize=None)
def _build_sc_kernel(N, L, D, V):
    TG = LANES
    n_per_w = N // NW
    n_groups = n_per_w // TG
    assert n_per_w * NW == N and n_groups * TG == n_per_w

    mesh = plsc.VectorSubcoreMesh(core_axis_name="c", subcore_axis_name="s")

    @functools.partial(
        pl.kernel,
        mesh=mesh,
        compiler_params=pltpu.CompilerParams(needs_layout_passes=False),
        out_type=jax.ShapeDtypeStruct((N,), jnp.float32),
        scratch_types=[
            pltpu.VMEM((L * TG,), jnp.int32),
            pltpu.VMEM((L * TG,), jnp.float32),
            pltpu.VMEM((D * TG,), jnp.float32),
            pltpu.VMEM((L * TG, D), jnp.float32),
            pltpu.VMEM((TG,), jnp.float32),
            pltpu.SemaphoreType.DMA,
        ],
    )
    def body(xf, zf, hf, w, out, xv, zv, hv, rows, outv, sem):
        wid = lax.axis_index("c") * NS + lax.axis_index("s")
        lane = lax.iota(jnp.int32, LANES)

        def group(g, carry):
            tb = wid * n_per_w + g * TG
            pltpu.sync_copy(xf.at[pl.ds(tb * L, L * TG)], xv)
            pltpu.sync_copy(zf.at[pl.ds(tb * L, L * TG)], zv)
            pltpu.sync_copy(hf.at[pl.ds(tb * D, D * TG)], hv)
            descs = [
                pltpu.async_copy(
                    w.at[xv.at[pl.ds(l * TG, TG)]],
                    rows.at[pl.ds(l * TG, TG)],
                    sem,
                )
                for l in range(L)
            ]
            for dsc in descs:
                dsc.wait()

            def dstep(d, accs):
                dspl = jnp.full((LANES,), d, dtype=jnp.int32)
                hvec = plsc.load_gather(hv, [lane + d * TG])
                return tuple(
                    accs[l] + plsc.load_gather(rows, [lane + l * TG, dspl]) * hvec
                    for l in range(L)
                )

            zero = jnp.zeros((LANES,), jnp.float32)
            accs = lax.fori_loop(0, D, dstep, tuple(zero for _ in range(L)))

            prod = jnp.ones((LANES,), jnp.float32)
            for l in range(L):
                t = accs[l] * zv[pl.ds(l * TG, TG)]
                y = 1.0 / (1.0 + jnp.exp(-t))
                y = jnp.where(xv[pl.ds(l * TG, TG)] != 0, y, 1.0)
                prod = prod * y
            outv[...] = prod
            pltpu.sync_copy(outv, out.at[pl.ds(tb, TG)])
            return carry

        lax.fori_loop(0, n_groups, group, 0)

    return body


def kernel(x, z, h, w):
    B, T, L = x.shape
    D = h.shape[-1]
    N = B * T
    NG = N // LANES
    xf = x.reshape(NG, LANES, L).transpose(0, 2, 1).reshape(-1).astype(jnp.int32)
    zf = z.reshape(NG, LANES, L).transpose(0, 2, 1).reshape(-1).astype(jnp.float32)
    hf = h.reshape(NG, LANES, D).transpose(0, 2, 1).reshape(-1).astype(jnp.float32)
    out = _build_sc_kernel(N, L, D, w.shape[0])(xf, zf, hf, w.astype(jnp.float32))
    return out.reshape(B, T)


# speedup vs baseline: 1.5771x; 1.1417x over previous
import functools

import jax
import jax.numpy as jnp
from jax import lax
from jax.experimental import pallas as pl
from jax.experimental.pallas import tpu as pltpu
from jax.experimental.pallas import tpu_sc as plsc

LANES = 16
NC, NS = 2, 16
NW = NC * NS


@functools.lru_cache(maxsize=None)
def _build_sc_kernel(N, L, D, V):
    TG = LANES
    GL = L * TG
    HL = D * TG
    RG = GL
    n_per_w = N // NW
    n_groups = n_per_w // TG
    n_pairs = n_groups // 2
    assert n_per_w * NW == N and n_pairs * 2 * TG == n_per_w

    mesh = plsc.VectorSubcoreMesh(core_axis_name="c", subcore_axis_name="s")

    @functools.partial(
        pl.kernel,
        mesh=mesh,
        compiler_params=pltpu.CompilerParams(needs_layout_passes=False),
        out_type=jax.ShapeDtypeStruct((N,), jnp.float32),
        scratch_types=[
            pltpu.VMEM((2 * GL,), jnp.int32),
            pltpu.VMEM((2 * GL,), jnp.float32),
            pltpu.VMEM((2 * HL,), jnp.float32),
            pltpu.VMEM((2 * RG, D), jnp.float32),
            pltpu.VMEM((GL,), jnp.float32),
            pltpu.VMEM((2 * TG,), jnp.float32),
            pltpu.SemaphoreType.DMA,
            pltpu.SemaphoreType.DMA,
            pltpu.SemaphoreType.DMA,
        ],
    )
    def body(xf, zf, hf, w, out, xs, zs, hs, rows, zmv, outv, sem_s, sem_r0, sem_r1):
        wid = lax.axis_index("c") * NS + lax.axis_index("s")
        lane = lax.iota(jnp.int32, LANES)
        base = wid * n_per_w

        def stage_copies(slot, tb):
            return (
                pltpu.make_async_copy(xf.at[pl.ds(tb * L, GL)],
                                      xs.at[pl.ds(slot * GL, GL)], sem_s),
                pltpu.make_async_copy(zf.at[pl.ds(tb * L, GL)],
                                      zs.at[pl.ds(slot * GL, GL)], sem_s),
                pltpu.make_async_copy(hf.at[pl.ds(tb * D, HL)],
                                      hs.at[pl.ds(slot * HL, HL)], sem_s),
            )

        def fire_stage(slot, tb):
            for c in stage_copies(slot, tb):
                c.start()

        def drain_stage(slot, tb):
            for c in stage_copies(slot, tb):
                c.wait()

        def row_copies(slot, tb):
            sem = sem_r0 if slot == 0 else sem_r1
            return [
                pltpu.make_async_copy(
                    w.at[xs.at[pl.ds(slot * GL + l * TG, TG)]],
                    rows.at[pl.ds(slot * RG + l * TG, TG)],
                    sem,
                )
                for l in range(L)
            ]

        def fire_rows(slot, tb):
            for c in row_copies(slot, tb):
                c.start()

        def drain_rows(slot, tb):
            for c in row_copies(slot, tb):
                c.wait()

        def prep_tail(slot):
            pc = jnp.ones((LANES,), jnp.float32)
            for l in range(L):
                xi = xs[pl.ds(slot * GL + l * TG, TG)]
                zl = zs[pl.ds(slot * GL + l * TG, TG)]
                m = xi != 0
                zmv[pl.ds(l * TG, TG)] = jnp.where(m, zl, 0.0)
                pc = pc * jnp.where(m, 1.0, 2.0)
            return pc

        def compute(slot, pc, out_half):
            UNROLL = 4
            rbase = slot * RG
            hbase = slot * HL

            def dstep(i, accs):
                accs = list(accs)
                for k in range(UNROLL):
                    d = i * UNROLL + k
                    dspl = jnp.full((LANES,), d, dtype=jnp.int32)
                    hvec = plsc.load_gather(hs, [lane + (d * TG + hbase)])
                    for l in range(L):
                        gvec = plsc.load_gather(rows, [lane + (rbase + l * TG), dspl])
                        accs[l] = accs[l] + gvec * hvec
                return tuple(accs)

            zero = jnp.zeros((LANES,), jnp.float32)
            accs = lax.fori_loop(0, D // UNROLL, dstep,
                                 tuple(zero for _ in range(L)))

            prod = pc
            for l in range(L):
                t = accs[l] * zmv[pl.ds(l * TG, TG)]
                prod = prod * (1.0 / (1.0 + jnp.exp(-t)))
            outv[pl.ds(out_half * TG, TG)] = prod

        fire_stage(0, base)
        drain_stage(0, base)
        fire_rows(0, base)
        fire_stage(1, base + TG)

        def pair(g2, carry):
            tb0 = base + g2 * (2 * TG)
            tb1 = tb0 + TG
            tb2 = tb0 + 2 * TG
            not_last = g2 < n_pairs - 1

            drain_stage(1, tb1)
            fire_rows(1, tb1)
            pc = prep_tail(0)
            drain_rows(0, tb0)
            compute(0, pc, 0)

            @pl.when(not_last)
            def _():
                fire_stage(0, tb2)

            @pl.when(not_last)
            def _():
                drain_stage(0, tb2)
                fire_rows(0, tb2)

            pc = prep_tail(1)
            drain_rows(1, tb1)
            compute(1, pc, 1)

            @pl.when(not_last)
            def _():
                fire_stage(1, tb2 + TG)

            pltpu.sync_copy(outv, out.at[pl.ds(tb0, 2 * TG)])
            return carry

        lax.fori_loop(0, n_pairs, pair, 0)

    return body


def kernel(x, z, h, w):
    B, T, L = x.shape
    D = h.shape[-1]
    N = B * T
    NG = N // LANES
    xf = x.reshape(NG, LANES, L).transpose(0, 2, 1).reshape(-1).astype(jnp.int32)
    zf = z.reshape(NG, LANES, L).transpose(0, 2, 1).reshape(-1).astype(jnp.float32)
    hf = h.reshape(NG, LANES, D).transpose(0, 2, 1).reshape(-1).astype(jnp.float32)
    out = _build_sc_kernel(N, L, D, w.shape[0])(xf, zf, hf, w.astype(jnp.float32))
    return out.reshape(B, T)


# speedup vs baseline: 6.7610x; 4.2869x over previous
import functools

import jax
import jax.numpy as jnp
from jax import lax
from jax.experimental import pallas as pl
from jax.experimental.pallas import tpu as pltpu
from jax.experimental.pallas import tpu_sc as plsc

LANES = 16
NC, NS = 2, 16
NW = NC * NS


@functools.lru_cache(maxsize=None)
def _build_sc_kernel(N, L, D, V):
    TG = LANES
    GL = L * TG
    HL = D * TG
    RG = GL
    n_per_w = N // NW
    n_groups = n_per_w // TG
    n_pairs = n_groups // 2
    assert n_per_w * NW == N and n_pairs * 2 * TG == n_per_w

    mesh = plsc.VectorSubcoreMesh(core_axis_name="c", subcore_axis_name="s")

    @functools.partial(
        pl.kernel,
        mesh=mesh,
        compiler_params=pltpu.CompilerParams(
            needs_layout_passes=False),
        out_type=jax.ShapeDtypeStruct((N,), jnp.float32),
        scratch_types=[
            pltpu.VMEM((2 * GL,), jnp.int32),
            pltpu.VMEM((2 * GL,), jnp.float32),
            pltpu.VMEM((2 * HL,), jnp.float32),
            pltpu.VMEM((2 * RG, D), jnp.float32),
            pltpu.VMEM((GL,), jnp.float32),
            pltpu.VMEM((2 * TG,), jnp.float32),
            pltpu.SemaphoreType.DMA,
            pltpu.SemaphoreType.DMA,
            pltpu.SemaphoreType.DMA,
        ],
    )
    def body(xf, zf, hf, w, out, xs, zs, hs, rows, zmv, outv, sem_s, sem_r0, sem_r1):
        wid = lax.axis_index("c") * NS + lax.axis_index("s")
        lane = lax.iota(jnp.int32, LANES)
        base = wid * n_per_w

        def stage_copies(slot, tb):
            return (
                pltpu.make_async_copy(xf.at[pl.ds(tb * L, GL)],
                                      xs.at[pl.ds(slot * GL, GL)], sem_s),
                pltpu.make_async_copy(zf.at[pl.ds(tb * L, GL)],
                                      zs.at[pl.ds(slot * GL, GL)], sem_s),
                pltpu.make_async_copy(hf.at[pl.ds(tb * D, HL)],
                                      hs.at[pl.ds(slot * HL, HL)], sem_s),
            )

        def fire_stage(slot, tb):
            for c in stage_copies(slot, tb):
                c.start()

        def drain_stage(slot, tb):
            for c in stage_copies(slot, tb):
                c.wait()

        def row_copies(slot, tb):
            sem = sem_r0 if slot == 0 else sem_r1
            return [
                pltpu.make_async_copy(
                    w.at[xs.at[pl.ds(slot * GL + l * TG, TG)]],
                    rows.at[pl.ds(slot * RG + l * TG, TG)],
                    sem,
                )
                for l in range(L)
            ]

        def fire_rows(slot, tb):
            for c in row_copies(slot, tb):
                c.start()

        def drain_rows(slot, tb):
            for c in row_copies(slot, tb):
                c.wait()

        def prep_tail(slot):
            pc = jnp.ones((LANES,), jnp.float32)
            for l in range(L):
                xi = xs[pl.ds(slot * GL + l * TG, TG)]
                zl = zs[pl.ds(slot * GL + l * TG, TG)]
                m = xi != 0
                zmv[pl.ds(l * TG, TG)] = jnp.where(m, zl, 0.0)
                pc = pc * jnp.where(m, 1.0, 2.0)
            return pc

        def compute(slot, pc, out_half):
            rbase = slot * RG
            hbase = slot * HL
            perms = [jnp.bitwise_xor(lane, k) for k in (1, 2, 4, 8)]

            def tstep(tp, dots):
                hvs = [hs[pl.ds(hbase + tp * D + j * LANES, LANES)]
                       for j in range(D // LANES)]
                is_tp = lane == tp
                dots = list(dots)
                for l in range(L):
                    r = rbase + l * TG + tp
                    acc = rows[r, pl.ds(0, LANES)] * hvs[0]
                    for j in range(1, D // LANES):
                        acc = acc + rows[r, pl.ds(j * LANES, LANES)] * hvs[j]
                    for p in perms:
                        acc = acc + jnp.take_along_axis(
                            acc, p, axis=0, mode="promise_in_bounds")
                    dots[l] = jnp.where(is_tp, acc, dots[l])
                return tuple(dots)

            zero = jnp.zeros((LANES,), jnp.float32)
            dots = lax.fori_loop(0, TG, tstep, tuple(zero for _ in range(L)))

            prod = pc
            for l in range(L):
                t = dots[l] * zmv[pl.ds(l * TG, TG)]
                prod = prod * (1.0 / (1.0 + jnp.exp(-t)))
            outv[pl.ds(out_half * TG, TG)] = prod

        fire_stage(0, base)
        drain_stage(0, base)
        fire_rows(0, base)
        fire_stage(1, base + TG)

        def pair(g2, carry):
            tb0 = base + g2 * (2 * TG)
            tb1 = tb0 + TG
            tb2 = tb0 + 2 * TG
            not_last = g2 < n_pairs - 1

            drain_stage(1, tb1)
            fire_rows(1, tb1)
            pc = prep_tail(0)
            drain_rows(0, tb0)
            compute(0, pc, 0)

            @pl.when(not_last)
            def _():
                fire_stage(0, tb2)

            @pl.when(not_last)
            def _():
                drain_stage(0, tb2)
                fire_rows(0, tb2)

            pc = prep_tail(1)
            drain_rows(1, tb1)
            compute(1, pc, 1)

            @pl.when(not_last)
            def _():
                fire_stage(1, tb2 + TG)

            pltpu.sync_copy(outv, out.at[pl.ds(tb0, 2 * TG)])
            return carry

        lax.fori_loop(0, n_pairs, pair, 0)

    return body


def kernel(x, z, h, w):
    B, T, L = x.shape
    D = h.shape[-1]
    N = B * T
    NG = N // LANES
    xf = x.reshape(NG, LANES, L).transpose(0, 2, 1).reshape(-1).astype(jnp.int32)
    zf = z.reshape(NG, LANES, L).transpose(0, 2, 1).reshape(-1).astype(jnp.float32)
    hf = h.reshape(-1).astype(jnp.float32)
    out = _build_sc_kernel(N, L, D, w.shape[0])(xf, zf, hf, w.astype(jnp.float32))
    return out.reshape(B, T)
